# baseline (speedup 1.0000x reference)
import functools

import jax
import jax.numpy as jnp
from jax import lax
from jax.experimental import pallas as pl
from jax.experimental.pallas import tpu as pltpu
from jax.experimental.pallas import tpu_sc as plsc

T = 8192
C = 8192
D = 256

BT = 2048
BC = 2048


def _argmin_body(xsq_ref, x2_ref, w_ref, wsq_ref, col_ref, out_ref, min_s, arg_s):
    j = pl.program_id(1)
    nj = pl.num_programs(1)

    @pl.when(j == 0)
    def _init():
        min_s[...] = jnp.full((BT, 1), jnp.inf, jnp.float32)
        arg_s[...] = jnp.zeros((BT, 1), jnp.int32)

    mm2 = jnp.dot(x2_ref[...], w_ref[...].T, preferred_element_type=jnp.float32)
    d = (xsq_ref[...] - mm2) + wsq_ref[...]

    bmin = jnp.min(d, axis=1, keepdims=True)
    barg = jnp.argmin(d, axis=1).astype(jnp.int32)[:, None] + j * BC

    bmin_r = bmin.astype(jnp.bfloat16).astype(jnp.float32)
    better = bmin < min_s[...]
    arg_s[...] = jnp.where(better, barg, arg_s[...])
    min_s[...] = jnp.where(better, bmin_r, min_s[...])

    @pl.when(j == nj - 1)
    def _emit():
        out_ref[...] = arg_s[...][:, 0]


_argmin_call = pl.pallas_call(
    _argmin_body,
    grid=(T // BT, C // BC),
    in_specs=[
        pl.BlockSpec((BT, 1), lambda i, j: (i, 0)),
        pl.BlockSpec((BT, D), lambda i, j: (i, 0)),
        pl.BlockSpec((BC, D), lambda i, j: (j, 0)),
        pl.BlockSpec((1, BC), lambda i, j: (0, j)),
        pl.BlockSpec((1, BC), lambda i, j: (0, 0)),
    ],
    out_specs=pl.BlockSpec((BT,), lambda i, j: (i,)),
    out_shape=jax.ShapeDtypeStruct((T,), jnp.int32),
    scratch_shapes=[
        pltpu.VMEM((BT, 1), jnp.float32),
        pltpu.VMEM((BT, 1), jnp.int32),
    ],
)


@functools.cache
def _make_sc_gather():
    info = plsc.get_sparse_core_info()
    nw = info.num_cores * info.num_subcores
    b_per_w = T // nw
    mesh = plsc.VectorSubcoreMesh(core_axis_name="c", subcore_axis_name="s")

    @functools.partial(
        pl.kernel,
        mesh=mesh,
        out_type=jax.ShapeDtypeStruct((T, D), jnp.float32),
        scratch_types=[
            pltpu.VMEM((b_per_w,), jnp.int32),
            pltpu.VMEM((b_per_w, D), jnp.float32),
            pltpu.SemaphoreType.DMA,
        ],
    )
    def gather_k(table_hbm, idx_hbm, out_hbm, idx_v, rows_v, sem):
        wid = lax.axis_index("s") * info.num_cores + lax.axis_index("c")
        base = wid * b_per_w
        pltpu.sync_copy(idx_hbm.at[pl.ds(base, b_per_w)], idx_v)
        pltpu.async_copy(table_hbm.at[idx_v], rows_v, sem).wait()
        pltpu.sync_copy(rows_v, out_hbm.at[pl.ds(base, b_per_w)])

    return gather_k


def kernel(inputs, weight):
    x = inputs.reshape(-1, D)
    xsq = jnp.sum(x ** 2, axis=1, keepdims=True)
    wsq = jnp.sum(weight ** 2, axis=1)[None, :]
    x2b = (2.0 * x).astype(jnp.bfloat16)
    wb = weight.astype(jnp.bfloat16)
    col = lax.broadcasted_iota(jnp.int32, (1, BC), 1)
    indices = _argmin_call(xsq, x2b, wb, wsq, col)
    quantized = _make_sc_gather()(weight, indices).reshape(inputs.shape)
    return (quantized, indices)


# speedup vs baseline: 1.1984x; 1.1984x over previous
import functools

import jax
import jax.numpy as jnp
from jax import lax
from jax.experimental import pallas as pl
from jax.experimental.pallas import tpu as pltpu
from jax.experimental.pallas import tpu_sc as plsc

T = 8192
C = 8192
D = 256

BT = 2048
BC = 2048


def _argmin_body(xsq_ref, x2_ref, w_ref, wsq_ref, col_ref, out_ref, min_s, arg_s):
    j = pl.program_id(1)
    nj = pl.num_programs(1)

    @pl.when(j == 0)
    def _init():
        min_s[...] = jnp.full((BT, 1), jnp.inf, jnp.float32)
        arg_s[...] = jnp.zeros((BT, 1), jnp.int32)

    mm2 = jnp.dot(x2_ref[...], w_ref[...].T, preferred_element_type=jnp.float32)
    d = (xsq_ref[...] - mm2) + wsq_ref[...]

    bmin = jnp.min(d, axis=1, keepdims=True)
    barg = jnp.min(jnp.where(d == bmin, col_ref[...], C), axis=1, keepdims=True) + j * BC

    bmin_r = bmin.astype(jnp.bfloat16).astype(jnp.float32)
    better = bmin < min_s[...]
    arg_s[...] = jnp.where(better, barg, arg_s[...])
    min_s[...] = jnp.where(better, bmin_r, min_s[...])

    @pl.when(j == nj - 1)
    def _emit():
        out_ref[...] = arg_s[...][:, 0]


_argmin_call = pl.pallas_call(
    _argmin_body,
    grid=(T // BT, C // BC),
    in_specs=[
        pl.BlockSpec((BT, 1), lambda i, j: (i, 0)),
        pl.BlockSpec((BT, D), lambda i, j: (i, 0)),
        pl.BlockSpec((BC, D), lambda i, j: (j, 0)),
        pl.BlockSpec((1, BC), lambda i, j: (0, j)),
        pl.BlockSpec((1, BC), lambda i, j: (0, 0)),
    ],
    out_specs=pl.BlockSpec((BT,), lambda i, j: (i,)),
    out_shape=jax.ShapeDtypeStruct((T,), jnp.int32),
    scratch_shapes=[
        pltpu.VMEM((BT, 1), jnp.float32),
        pltpu.VMEM((BT, 1), jnp.int32),
    ],
)


@functools.cache
def _make_sc_gather():
    info = plsc.get_sparse_core_info()
    nw = info.num_cores * info.num_subcores
    b_per_w = T // nw
    mesh = plsc.VectorSubcoreMesh(core_axis_name="c", subcore_axis_name="s")

    @functools.partial(
        pl.kernel,
        mesh=mesh,
        out_type=jax.ShapeDtypeStruct((T, D), jnp.float32),
        scratch_types=[
            pltpu.VMEM((b_per_w,), jnp.int32),
            pltpu.VMEM((b_per_w, D), jnp.float32),
            pltpu.SemaphoreType.DMA,
        ],
    )
    def gather_k(table_hbm, idx_hbm, out_hbm, idx_v, rows_v, sem):
        wid = lax.axis_index("s") * info.num_cores + lax.axis_index("c")
        base = wid * b_per_w
        pltpu.sync_copy(idx_hbm.at[pl.ds(base, b_per_w)], idx_v)
        pltpu.async_copy(table_hbm.at[idx_v], rows_v, sem).wait()
        pltpu.sync_copy(rows_v, out_hbm.at[pl.ds(base, b_per_w)])

    return gather_k


def kernel(inputs, weight):
    x = inputs.reshape(-1, D)
    xsq = jnp.sum(x ** 2, axis=1, keepdims=True)
    wsq = jnp.sum(weight ** 2, axis=1)[None, :]
    x2b = (2.0 * x).astype(jnp.bfloat16)
    wb = weight.astype(jnp.bfloat16)
    col = lax.broadcasted_iota(jnp.int32, (1, BC), 1)
    indices = _argmin_call(xsq, x2b, wb, wsq, col)
    quantized = _make_sc_gather()(weight, indices).reshape(inputs.shape)
    return (quantized, indices)
